# Initial kernel scaffold; baseline (speedup 1.0000x reference)
import jax
import jax.numpy as jnp
from jax.experimental import pallas as pl


def kernel(x, edge_index, batch, W0, W1, W2, conv_bias, lin_w, lin_b):
    raise NotImplementedError("write your pallas kernel here")


# speedup vs baseline: 6.5060x; 6.5060x over previous
import functools

import jax
import jax.numpy as jnp
from jax import lax
from jax.experimental import pallas as pl
from jax.experimental.pallas import tpu as pltpu
from jax.experimental.pallas import tpu_sc as plsc

N = 10000
E = 320000
D = 128
NCLS = 10

NC = 2
NS = 16
NW = NC * NS
CH = 80
EPT = E // NW
EPC = E // NS
NROW = 10240
RPT = NROW // NS
NPAD = 10240
ZB = 640

_mesh = plsc.VectorSubcoreMesh(core_axis_name="c", subcore_axis_name="s")


def _zero_fill(zrows_v, zvec_v):
    zeros16 = jnp.zeros((16,), jnp.float32)

    def zr(i, _):
        for cc in range(8):
            zrows_v[i, pl.ds(cc * 16, 16)] = zeros16
        return 0

    lax.fori_loop(0, zrows_v.shape[0], zr, 0)

    def zv(i, _):
        zvec_v[pl.ds(i * 16, 16)] = zeros16
        return 0

    lax.fori_loop(0, zvec_v.shape[0] // 16, zv, 0)


def _scale_rows(rows_v, normv):
    def body(j16, _):
        nv16 = normv[pl.ds(j16 * 16, 16)]
        for k in range(16):
            nv = nv16[k]
            j = j16 * 16 + k
            for cc in range(8):
                sl = pl.ds(cc * 16, 16)
                rows_v[j, sl] = rows_v[j, sl] * nv
        return 0

    lax.fori_loop(0, CH // 16, body, 0)


@functools.partial(
    pl.kernel,
    out_type=(
        jax.ShapeDtypeStruct((NC, NROW, D), jnp.float32),
        jax.ShapeDtypeStruct((E,), jnp.float32),
    ),
    mesh=_mesh,
    compiler_params=pltpu.CompilerParams(needs_layout_passes=False),
    scratch_types=[
        pltpu.VMEM((128, D), jnp.float32),
        pltpu.VMEM((ZB,), jnp.float32),
        pltpu.VMEM((NPAD,), jnp.float32),
        pltpu.VMEM((CH,), jnp.int32),
        pltpu.VMEM((CH,), jnp.int32),
        pltpu.VMEM((CH,), jnp.float32),
        pltpu.VMEM((CH, D), jnp.float32),
        pltpu.VMEM_SHARED((NROW, D), jnp.float32),
        pltpu.VMEM_SHARED((NPAD,), jnp.float32),
        pltpu.SemaphoreType.DMA,
    ],
)
def _sc_prop1(src_hbm, dst_hbm, x_hbm, p1_hbm, norm_hbm,
              zrows_v, zvec_v, dinv_v, srcv, dstv, normv, rows_v,
              acc_sh, deg_sh, sem):
    c = lax.axis_index("c")
    s = lax.axis_index("s")
    wid = c * NS + s

    _zero_fill(zrows_v, zvec_v)
    pltpu.sync_copy(zvec_v, deg_sh.at[pl.ds(s * ZB, ZB)])
    for k in range(5):
        pltpu.sync_copy(zrows_v, acc_sh.at[pl.ds(s * RPT + k * 128, 128)])
    plsc.subcore_barrier()

    def deg_body(i, _):
        base = s * EPC + i * CH
        pltpu.sync_copy(src_hbm.at[pl.ds(base, CH)], srcv)
        pltpu.sync_copy(dst_hbm.at[pl.ds(base, CH)], dstv)
        for j in range(CH // 16):
            sl = pl.ds(j * 16, 16)
            sv = srcv[sl]
            dv = dstv[sl]
            normv[sl] = jnp.where(sv != dv, 1.0, 0.0).astype(jnp.float32)
        pltpu.sync_copy(normv, deg_sh.at[srcv], add=True)
        return 0

    lax.fori_loop(0, EPC // CH, deg_body, 0)
    plsc.subcore_barrier()

    pltpu.sync_copy(deg_sh, dinv_v)

    def dinv_body(i, _):
        sl = pl.ds(i * 16, 16)
        d = dinv_v[sl]
        ih = lax.bitcast_convert_type(d, jnp.int32)
        y = lax.bitcast_convert_type(jnp.int32(0x5F3759DF) - (ih >> 1), jnp.float32)
        for _r in range(3):
            y = y * (1.5 - 0.5 * d * y * y)
        dinv_v[sl] = jnp.where(d > 0.0, y, 0.0)
        return 0

    lax.fori_loop(0, NPAD // 16, dinv_body, 0)

    def prop_body(i, _):
        base = wid * EPT + i * CH
        pltpu.sync_copy(src_hbm.at[pl.ds(base, CH)], srcv)
        pltpu.sync_copy(dst_hbm.at[pl.ds(base, CH)], dstv)
        for j in range(CH // 16):
            sl = pl.ds(j * 16, 16)
            sv = srcv[sl]
            dv = dstv[sl]
            a = plsc.load_gather(dinv_v, [sv])
            b = plsc.load_gather(dinv_v, [dv])
            normv[sl] = jnp.where(sv != dv, -(a * b), 0.0).astype(jnp.float32)
        pltpu.async_copy(x_hbm.at[srcv], rows_v, sem).wait()
        _scale_rows(rows_v, normv)
        pltpu.sync_copy(rows_v, acc_sh.at[dstv], add=True)
        pltpu.sync_copy(normv, norm_hbm.at[pl.ds(base, CH)])
        return 0

    lax.fori_loop(0, EPT // CH, prop_body, 0)
    plsc.subcore_barrier()

    pltpu.sync_copy(acc_sh.at[pl.ds(s * RPT, RPT)], p1_hbm.at[c, pl.ds(s * RPT, RPT)])


@functools.partial(
    pl.kernel,
    out_type=jax.ShapeDtypeStruct((NC, NROW, D), jnp.float32),
    mesh=_mesh,
    compiler_params=pltpu.CompilerParams(needs_layout_passes=False),
    scratch_types=[
        pltpu.VMEM((128, D), jnp.float32),
        pltpu.VMEM((CH,), jnp.int32),
        pltpu.VMEM((CH,), jnp.int32),
        pltpu.VMEM((CH,), jnp.float32),
        pltpu.VMEM((CH, D), jnp.float32),
        pltpu.VMEM_SHARED((NROW, D), jnp.float32),
        pltpu.SemaphoreType.DMA,
    ],
)
def _sc_prop2(src_hbm, dst_hbm, norm_hbm, h_hbm, p2_hbm,
              zrows_v, srcv, dstv, normv, rows_v, acc_sh, sem):
    c = lax.axis_index("c")
    s = lax.axis_index("s")
    wid = c * NS + s

    zeros16 = jnp.zeros((16,), jnp.float32)

    def zr(i, _):
        for cc in range(8):
            zrows_v[i, pl.ds(cc * 16, 16)] = zeros16
        return 0

    lax.fori_loop(0, 128, zr, 0)
    for k in range(5):
        pltpu.sync_copy(zrows_v, acc_sh.at[pl.ds(s * RPT + k * 128, 128)])
    plsc.subcore_barrier()

    def prop_body(i, _):
        base = wid * EPT + i * CH
        pltpu.sync_copy(src_hbm.at[pl.ds(base, CH)], srcv)
        pltpu.sync_copy(dst_hbm.at[pl.ds(base, CH)], dstv)
        pltpu.sync_copy(norm_hbm.at[pl.ds(base, CH)], normv)
        pltpu.async_copy(h_hbm.at[srcv], rows_v, sem).wait()
        _scale_rows(rows_v, normv)
        pltpu.sync_copy(rows_v, acc_sh.at[dstv], add=True)
        return 0

    lax.fori_loop(0, EPT // CH, prop_body, 0)
    plsc.subcore_barrier()

    pltpu.sync_copy(acc_sh.at[pl.ds(s * RPT, RPT)], p2_hbm.at[c, pl.ds(s * RPT, RPT)])


_BM = 1000


def _tc_mid_body(x_ref, pp_ref, w0_ref, w1_ref, w2_ref, b_ref, p1_ref, out0_ref):
    pp = pp_ref[...]
    p1 = pp[0] + pp[1]
    p1_ref[...] = p1
    w02 = w0_ref[...] - w2_ref[...]
    out0_ref[...] = (
        jnp.dot(x_ref[...], w02, preferred_element_type=jnp.float32)
        + jnp.dot(p1, w1_ref[...], preferred_element_type=jnp.float32)
        + b_ref[...]
    )


def _tc_mid(x, p1parts, W0, W1, W2, bias2d):
    return pl.pallas_call(
        _tc_mid_body,
        grid=(N // _BM,),
        in_specs=[
            pl.BlockSpec((_BM, D), lambda i: (i, 0)),
            pl.BlockSpec((NC, _BM, D), lambda i: (0, i, 0)),
            pl.BlockSpec((D, D), lambda i: (0, 0)),
            pl.BlockSpec((D, D), lambda i: (0, 0)),
            pl.BlockSpec((D, D), lambda i: (0, 0)),
            pl.BlockSpec((1, D), lambda i: (0, 0)),
        ],
        out_specs=[
            pl.BlockSpec((_BM, D), lambda i: (i, 0)),
            pl.BlockSpec((_BM, D), lambda i: (i, 0)),
        ],
        out_shape=[
            jax.ShapeDtypeStruct((N, D), jnp.float32),
            jax.ShapeDtypeStruct((N, D), jnp.float32),
        ],
    )(x, p1parts, W0, W1, W2, bias2d)


def _tc_final_body(out0_ref, pp_ref, w2_ref, lw_ref, lb_ref, logits_ref):
    i = pl.program_id(0)
    pp = pp_ref[...]
    p2 = pp[0] + pp[1]
    out = out0_ref[...] + 2.0 * jnp.dot(p2, w2_ref[...],
                                        preferred_element_type=jnp.float32)
    h = jnp.maximum(out, 0.0)
    contrib = jnp.sum(lw_ref[...] * h[None, :, :], axis=(1, 2))

    @pl.when(i == 0)
    def _():
        logits_ref[...] = lb_ref[...]

    logits_ref[...] += contrib[None, :]


def _tc_final(out0, p2parts, W2, lin_w3, lin_b2d):
    return pl.pallas_call(
        _tc_final_body,
        grid=(N // _BM,),
        in_specs=[
            pl.BlockSpec((_BM, D), lambda i: (i, 0)),
            pl.BlockSpec((NC, _BM, D), lambda i: (0, i, 0)),
            pl.BlockSpec((D, D), lambda i: (0, 0)),
            pl.BlockSpec((NCLS, _BM, D), lambda i: (0, i, 0)),
            pl.BlockSpec((1, NCLS), lambda i: (0, 0)),
        ],
        out_specs=pl.BlockSpec((1, NCLS), lambda i: (0, 0)),
        out_shape=jax.ShapeDtypeStruct((1, NCLS), jnp.float32),
    )(out0, p2parts, W2, lin_w3, lin_b2d)


def kernel(x, edge_index, batch, W0, W1, W2, conv_bias, lin_w, lin_b):
    src = edge_index[0]
    dst = edge_index[1]

    p1parts, norm = _sc_prop1(src, dst, x)
    p1, out0 = _tc_mid(x, p1parts, W0, W1, W2, conv_bias.reshape(1, D))
    p2parts = _sc_prop2(src, dst, norm, p1)
    logits = _tc_final(out0, p2parts, W2, lin_w.reshape(NCLS, N, D),
                       lin_b.reshape(1, NCLS))
    return logits


# speedup vs baseline: 15.3150x; 2.3540x over previous
import functools

import jax
import jax.numpy as jnp
from jax import lax
from jax.experimental import pallas as pl
from jax.experimental.pallas import tpu as pltpu
from jax.experimental.pallas import tpu_sc as plsc

N = 10000
E = 320000
D = 128
NCLS = 10

NC = 2
NS = 16
NW = NC * NS
CH = 80
GRP = 5
EPT = E // NW
NG = EPT // (CH * GRP)
PGRP = 2
PNG = (EPT // CH) // PGRP
NROW = 10240
RPT = NROW // NS
ZB = 640
DUMMY = 10016

_mesh = plsc.VectorSubcoreMesh(core_axis_name="c", subcore_axis_name="s")
_sc_params = pltpu.CompilerParams(needs_layout_passes=False, disable_bounds_checks=True)


@functools.partial(
    pl.kernel,
    out_type=jax.ShapeDtypeStruct((NC, NROW), jnp.float32),
    mesh=_mesh,
    compiler_params=_sc_params,
    scratch_types=[
        pltpu.VMEM((ZB,), jnp.float32),
        pltpu.VMEM((CH,), jnp.float32),
        pltpu.VMEM((2, GRP, CH), jnp.int32),
        pltpu.VMEM_SHARED((NROW,), jnp.float32),
        pltpu.SemaphoreType.DMA,
        pltpu.SemaphoreType.DMA,
    ],
)
def _sc_deg(srcp_hbm, deg_hbm,
            zvec_v, ones_v, srcv, deg_sh, isem, ssem):
    c = lax.axis_index("c")
    s = lax.axis_index("s")
    base_t = c * (E // NC) + s * EPT

    zeros16 = jnp.zeros((16,), jnp.float32)
    ones16 = jnp.ones((16,), jnp.float32)

    def zfill(i, _):
        zvec_v[pl.ds(i * 16, 16)] = zeros16
        return 0

    lax.fori_loop(0, ZB // 16, zfill, 0)
    for j in range(CH // 16):
        ones_v[pl.ds(j * 16, 16)] = ones16

    pltpu.sync_copy(zvec_v, deg_sh.at[pl.ds(s * ZB, ZB)])

    def fire_idx(o, slot):
        for b in range(GRP):
            base = base_t + (o * GRP + b) * CH
            pltpu.async_copy(srcp_hbm.at[pl.ds(base, CH)], srcv.at[slot, b], isem)

    def drain_idx(o, slot):
        for b in range(GRP):
            base = base_t + (o * GRP + b) * CH
            pltpu.make_async_copy(srcp_hbm.at[pl.ds(base, CH)], srcv.at[slot, b], isem).wait()

    fire_idx(0, 0)
    plsc.subcore_barrier()

    def body(o, _):
        g = lax.rem(o, 2)
        g1 = lax.rem(o + 1, 2)
        drain_idx(o, g)

        @pl.when(o > 0)
        def _():
            for b in range(GRP):
                pltpu.make_async_copy(ones_v, deg_sh.at[srcv.at[g1, b]], ssem).wait()

        @pl.when(o < NG - 1)
        def _():
            fire_idx(o + 1, g1)

        for b in range(GRP):
            pltpu.async_copy(ones_v, deg_sh.at[srcv.at[g, b]], ssem, add=True)
        return 0

    lax.fori_loop(0, NG, body, 0)
    for b in range(GRP):
        pltpu.make_async_copy(ones_v, deg_sh.at[srcv.at[0, b]], ssem).wait()
    plsc.subcore_barrier()
    pltpu.sync_copy(deg_sh.at[pl.ds(s * ZB, ZB)], deg_hbm.at[c, pl.ds(s * ZB, ZB)])


@functools.partial(
    pl.kernel,
    out_type=jax.ShapeDtypeStruct((NC, NROW, D), jnp.float32),
    mesh=_mesh,
    compiler_params=_sc_params,
    scratch_types=[
        pltpu.VMEM((32, D), jnp.float32),
        pltpu.VMEM((2, PGRP, CH), jnp.int32),
        pltpu.VMEM((2, PGRP, CH), jnp.int32),
        pltpu.VMEM((2, PGRP, CH, D), jnp.float32),
        pltpu.VMEM_SHARED((NROW, D), jnp.float32),
        pltpu.SemaphoreType.DMA,
        pltpu.SemaphoreType.DMA,
        pltpu.SemaphoreType.DMA,
    ],
)
def _sc_prop(src_hbm, dstp_hbm, tab_hbm, acc_hbm,
             zrows_v, srcv, dstv, rows_v, acc_sh, isem, gsem, ssem):
    c = lax.axis_index("c")
    s = lax.axis_index("s")
    wid = c * NS + s
    base_t = wid * EPT

    zeros16 = jnp.zeros((16,), jnp.float32)

    def zfill(i, _):
        for cc in range(8):
            zrows_v[i, pl.ds(cc * 16, 16)] = zeros16
        return 0

    lax.fori_loop(0, 32, zfill, 0)
    for k in range(20):
        pltpu.sync_copy(zrows_v, acc_sh.at[pl.ds(s * RPT + k * 32, 32)])

    def fire_idx(o, slot):
        for b in range(PGRP):
            base = base_t + (o * PGRP + b) * CH
            pltpu.async_copy(src_hbm.at[pl.ds(base, CH)], srcv.at[slot, b], isem)
            pltpu.async_copy(dstp_hbm.at[pl.ds(base, CH)], dstv.at[slot, b], isem)

    def drain_idx(o, slot):
        for b in range(PGRP):
            base = base_t + (o * PGRP + b) * CH
            pltpu.make_async_copy(src_hbm.at[pl.ds(base, CH)], srcv.at[slot, b], isem).wait()
            pltpu.make_async_copy(dstp_hbm.at[pl.ds(base, CH)], dstv.at[slot, b], isem).wait()

    fire_idx(0, 0)
    plsc.subcore_barrier()

    def body(o, _):
        g = lax.rem(o, 2)
        g1 = lax.rem(o + 1, 2)
        drain_idx(o, g)

        @pl.when(o > 0)
        def _():
            for b in range(PGRP):
                pltpu.make_async_copy(rows_v.at[g1, b], acc_sh.at[dstv.at[g1, b]], ssem).wait()

        for b in range(PGRP):
            pltpu.async_copy(tab_hbm.at[srcv.at[g, b]], rows_v.at[g, b], gsem)

        @pl.when(o < PNG - 1)
        def _():
            fire_idx(o + 1, g1)

        for b in range(PGRP):
            pltpu.make_async_copy(tab_hbm.at[srcv.at[g, b]], rows_v.at[g, b], gsem).wait()
        for b in range(PGRP):
            pltpu.async_copy(rows_v.at[g, b], acc_sh.at[dstv.at[g, b]], ssem, add=True)
        return 0

    lax.fori_loop(0, PNG, body, 0)
    gl = (PNG - 1) % 2
    for b in range(PGRP):
        pltpu.make_async_copy(rows_v.at[gl, b], acc_sh.at[dstv.at[gl, b]], ssem).wait()

    tbase = base_t + (PNG * PGRP) * CH
    pltpu.sync_copy(src_hbm.at[pl.ds(tbase, CH)], srcv.at[0, 0])
    pltpu.sync_copy(dstp_hbm.at[pl.ds(tbase, CH)], dstv.at[0, 0])
    pltpu.async_copy(tab_hbm.at[srcv.at[0, 0]], rows_v.at[0, 0], gsem).wait()
    pltpu.async_copy(rows_v.at[0, 0], acc_sh.at[dstv.at[0, 0]], ssem, add=True)
    pltpu.make_async_copy(rows_v.at[0, 0], acc_sh.at[dstv.at[0, 0]], ssem).wait()

    plsc.subcore_barrier()
    pltpu.sync_copy(acc_sh.at[pl.ds(s * RPT, RPT)], acc_hbm.at[c, pl.ds(s * RPT, RPT)])


EF = 2500


def _tc_fix_body(src_ref, dst_ref, srcp_ref, dstp_ref):
    sv = src_ref[...]
    dv = dst_ref[...]
    ne = sv != dv
    srcp_ref[...] = jnp.where(ne, sv, DUMMY)
    dstp_ref[...] = jnp.where(ne, dv, DUMMY)


def _tc_fix(src2d, dst2d):
    return pl.pallas_call(
        _tc_fix_body,
        out_shape=[
            jax.ShapeDtypeStruct((EF, 128), jnp.int32),
            jax.ShapeDtypeStruct((EF, 128), jnp.int32),
        ],
    )(src2d, dst2d)


def _tc_dinv_body(deg_ref, dinv_ref):
    deg = deg_ref[0:1, :] + deg_ref[1:2, :]
    r = lax.rsqrt(deg)
    dinv_ref[...] = jnp.where(deg > 0.0, r, 0.0)


def _tc_dinv(deg2):
    return pl.pallas_call(
        _tc_dinv_body,
        out_shape=jax.ShapeDtypeStruct((1, NROW), jnp.float32),
    )(deg2)


_BM = 1000


def _tc_xscale_body(x_ref, dinv_ref, xt_ref):
    xt_ref[...] = x_ref[...] * dinv_ref[...]


def _tc_xscale(x, dinv_col):
    return pl.pallas_call(
        _tc_xscale_body,
        grid=(N // _BM,),
        in_specs=[
            pl.BlockSpec((_BM, D), lambda i: (i, 0)),
            pl.BlockSpec((_BM, 1), lambda i: (i, 0)),
        ],
        out_specs=pl.BlockSpec((_BM, D), lambda i: (i, 0)),
        out_shape=jax.ShapeDtypeStruct((N, D), jnp.float32),
    )(x, dinv_col)


def _tc_mid_body(x_ref, u1_ref, dinv_ref, w0_ref, w1_ref, w2_ref, b_ref,
                 p1t_ref, out0_ref):
    dv = dinv_ref[...]
    u1 = u1_ref[0] + u1_ref[1]
    p1 = -dv * u1
    p1t_ref[...] = dv * p1
    w02 = w0_ref[...] - w2_ref[...]
    out0_ref[...] = (
        jnp.dot(x_ref[...], w02, preferred_element_type=jnp.float32)
        + jnp.dot(p1, w1_ref[...], preferred_element_type=jnp.float32)
        + b_ref[...]
    )


def _tc_mid(x, u1parts, dinv_col, W0, W1, W2, bias2d):
    return pl.pallas_call(
        _tc_mid_body,
        grid=(N // _BM,),
        in_specs=[
            pl.BlockSpec((_BM, D), lambda i: (i, 0)),
            pl.BlockSpec((NC, _BM, D), lambda i: (0, i, 0)),
            pl.BlockSpec((_BM, 1), lambda i: (i, 0)),
            pl.BlockSpec((D, D), lambda i: (0, 0)),
            pl.BlockSpec((D, D), lambda i: (0, 0)),
            pl.BlockSpec((D, D), lambda i: (0, 0)),
            pl.BlockSpec((1, D), lambda i: (0, 0)),
        ],
        out_specs=[
            pl.BlockSpec((_BM, D), lambda i: (i, 0)),
            pl.BlockSpec((_BM, D), lambda i: (i, 0)),
        ],
        out_shape=[
            jax.ShapeDtypeStruct((N, D), jnp.float32),
            jax.ShapeDtypeStruct((N, D), jnp.float32),
        ],
    )(x, u1parts, dinv_col, W0, W1, W2, bias2d)


def _tc_final_body(out0_ref, u2_ref, dinv_ref, w2_ref, lw_ref, lb_ref,
                   logits_ref):
    i = pl.program_id(0)
    u2 = u2_ref[0] + u2_ref[1]
    p2 = -dinv_ref[...] * u2
    out = out0_ref[...] + 2.0 * jnp.dot(p2, w2_ref[...],
                                        preferred_element_type=jnp.float32)
    h = jnp.maximum(out, 0.0)
    contrib = jnp.sum(lw_ref[...] * h[None, :, :], axis=(1, 2))

    @pl.when(i == 0)
    def _():
        logits_ref[...] = lb_ref[...]

    logits_ref[...] += contrib[None, :]


def _tc_final(out0, u2parts, dinv_col, W2, lin_w3, lin_b2d):
    return pl.pallas_call(
        _tc_final_body,
        grid=(N // _BM,),
        in_specs=[
            pl.BlockSpec((_BM, D), lambda i: (i, 0)),
            pl.BlockSpec((NC, _BM, D), lambda i: (0, i, 0)),
            pl.BlockSpec((_BM, 1), lambda i: (i, 0)),
            pl.BlockSpec((D, D), lambda i: (0, 0)),
            pl.BlockSpec((NCLS, _BM, D), lambda i: (0, i, 0)),
            pl.BlockSpec((1, NCLS), lambda i: (0, 0)),
        ],
        out_specs=pl.BlockSpec((1, NCLS), lambda i: (0, 0)),
        out_shape=jax.ShapeDtypeStruct((1, NCLS), jnp.float32),
    )(out0, u2parts, dinv_col, W2, lin_w3, lin_b2d)


def kernel(x, edge_index, batch, W0, W1, W2, conv_bias, lin_w, lin_b):
    src = edge_index[0]
    dst = edge_index[1]

    srcp2d, dstp2d = _tc_fix(src.reshape(EF, 128), dst.reshape(EF, 128))
    srcp = srcp2d.reshape(E)
    dstp = dstp2d.reshape(E)
    deg2 = _sc_deg(srcp)
    dinv_row = _tc_dinv(deg2)
    dinv_col = dinv_row.reshape(NROW, 1)
    xt = _tc_xscale(x, dinv_col)
    u1parts = _sc_prop(src, dstp, xt)
    p1t, out0 = _tc_mid(x, u1parts, dinv_col, W0, W1, W2,
                        conv_bias.reshape(1, D))
    u2parts = _sc_prop(src, dstp, p1t)
    logits = _tc_final(out0, u2parts, dinv_col, W2, lin_w.reshape(NCLS, N, D),
                       lin_b.reshape(1, NCLS))
    return logits


# speedup vs baseline: 17.1028x; 1.1167x over previous
import functools

import jax
import jax.numpy as jnp
from jax import lax
from jax.experimental import pallas as pl
from jax.experimental.pallas import tpu as pltpu
from jax.experimental.pallas import tpu_sc as plsc

N = 10000
E = 320000
D = 128
NCLS = 10

NC = 2
NS = 16
NW = NC * NS
CH = 80
GRP = 5
EPT = E // NW
NG = EPT // (CH * GRP)
PGRP = 2
PNG = (EPT // CH) // PGRP
NROW = 10240
RPT = NROW // NS
ZB = 640
DUMMY = 10016

_mesh = plsc.VectorSubcoreMesh(core_axis_name="c", subcore_axis_name="s")
_sc_params = pltpu.CompilerParams(needs_layout_passes=False, disable_bounds_checks=True)


@functools.partial(
    pl.kernel,
    out_type=jax.ShapeDtypeStruct((NC, NROW), jnp.float32),
    mesh=_mesh,
    compiler_params=_sc_params,
    scratch_types=[
        pltpu.VMEM((ZB,), jnp.float32),
        pltpu.VMEM((CH,), jnp.float32),
        pltpu.VMEM((2, GRP, CH), jnp.int32),
        pltpu.VMEM_SHARED((NROW,), jnp.float32),
        pltpu.SemaphoreType.DMA,
        pltpu.SemaphoreType.DMA,
    ],
)
def _sc_deg(srcp_hbm, deg_hbm,
            zvec_v, ones_v, srcv, deg_sh, isem, ssem):
    c = lax.axis_index("c")
    s = lax.axis_index("s")
    base_t = c * (E // NC) + s * EPT

    zeros16 = jnp.zeros((16,), jnp.float32)
    ones16 = jnp.ones((16,), jnp.float32)

    def zfill(i, _):
        zvec_v[pl.ds(i * 16, 16)] = zeros16
        return 0

    lax.fori_loop(0, ZB // 16, zfill, 0)
    for j in range(CH // 16):
        ones_v[pl.ds(j * 16, 16)] = ones16

    pltpu.sync_copy(zvec_v, deg_sh.at[pl.ds(s * ZB, ZB)])

    def fire_idx(o, slot):
        for b in range(GRP):
            base = base_t + (o * GRP + b) * CH
            pltpu.async_copy(srcp_hbm.at[pl.ds(base, CH)], srcv.at[slot, b], isem)

    def drain_idx(o, slot):
        for b in range(GRP):
            base = base_t + (o * GRP + b) * CH
            pltpu.make_async_copy(srcp_hbm.at[pl.ds(base, CH)], srcv.at[slot, b], isem).wait()

    fire_idx(0, 0)
    plsc.subcore_barrier()

    def body(o, _):
        g = lax.rem(o, 2)
        g1 = lax.rem(o + 1, 2)
        drain_idx(o, g)

        @pl.when(o > 0)
        def _():
            for b in range(GRP):
                pltpu.make_async_copy(ones_v, deg_sh.at[srcv.at[g1, b]], ssem).wait()

        @pl.when(o < NG - 1)
        def _():
            fire_idx(o + 1, g1)

        for b in range(GRP):
            pltpu.async_copy(ones_v, deg_sh.at[srcv.at[g, b]], ssem, add=True)
        return 0

    lax.fori_loop(0, NG, body, 0)
    for b in range(GRP):
        pltpu.make_async_copy(ones_v, deg_sh.at[srcv.at[0, b]], ssem).wait()
    plsc.subcore_barrier()
    pltpu.sync_copy(deg_sh.at[pl.ds(s * ZB, ZB)], deg_hbm.at[c, pl.ds(s * ZB, ZB)])


@functools.partial(
    pl.kernel,
    out_type=jax.ShapeDtypeStruct((NC, NROW, D), jnp.float32),
    mesh=_mesh,
    compiler_params=_sc_params,
    scratch_types=[
        pltpu.VMEM((32, D), jnp.float32),
        pltpu.VMEM((2, PGRP, CH), jnp.int32),
        pltpu.VMEM((2, PGRP, CH), jnp.int32),
        pltpu.VMEM((2, PGRP, CH, D), jnp.float32),
        pltpu.VMEM_SHARED((NROW, D), jnp.float32),
        pltpu.SemaphoreType.DMA,
        pltpu.SemaphoreType.DMA,
        pltpu.SemaphoreType.DMA,
    ],
)
def _sc_prop(src_hbm, dstp_hbm, tab_hbm, acc_hbm,
             zrows_v, srcv, dstv, rows_v, acc_sh, isem, gsem, ssem):
    c = lax.axis_index("c")
    s = lax.axis_index("s")
    wid = c * NS + s
    base_t = wid * EPT

    zeros16 = jnp.zeros((16,), jnp.float32)

    def zfill(i, _):
        for cc in range(8):
            zrows_v[i, pl.ds(cc * 16, 16)] = zeros16
        return 0

    lax.fori_loop(0, 32, zfill, 0)
    for k in range(20):
        pltpu.sync_copy(zrows_v, acc_sh.at[pl.ds(s * RPT + k * 32, 32)])

    def fire_idx(o, slot):
        for b in range(PGRP):
            base = base_t + (o * PGRP + b) * CH
            pltpu.async_copy(src_hbm.at[pl.ds(base, CH)], srcv.at[slot, b], isem)
            pltpu.async_copy(dstp_hbm.at[pl.ds(base, CH)], dstv.at[slot, b], isem)

    def drain_idx(o, slot):
        for b in range(PGRP):
            base = base_t + (o * PGRP + b) * CH
            pltpu.make_async_copy(src_hbm.at[pl.ds(base, CH)], srcv.at[slot, b], isem).wait()
            pltpu.make_async_copy(dstp_hbm.at[pl.ds(base, CH)], dstv.at[slot, b], isem).wait()

    fire_idx(0, 0)
    plsc.subcore_barrier()

    def body(o, _):
        g = lax.rem(o, 2)
        g1 = lax.rem(o + 1, 2)
        drain_idx(o, g)

        @pl.when(o > 0)
        def _():
            for b in range(PGRP):
                pltpu.make_async_copy(rows_v.at[g1, b], acc_sh.at[dstv.at[g1, b]], ssem).wait()

        for b in range(PGRP):
            pltpu.async_copy(tab_hbm.at[srcv.at[g, b]], rows_v.at[g, b], gsem)

        @pl.when(o < PNG - 1)
        def _():
            fire_idx(o + 1, g1)

        for b in range(PGRP):
            pltpu.make_async_copy(tab_hbm.at[srcv.at[g, b]], rows_v.at[g, b], gsem).wait()
        for b in range(PGRP):
            pltpu.async_copy(rows_v.at[g, b], acc_sh.at[dstv.at[g, b]], ssem, add=True)
        return 0

    lax.fori_loop(0, PNG, body, 0)
    gl = (PNG - 1) % 2
    for b in range(PGRP):
        pltpu.make_async_copy(rows_v.at[gl, b], acc_sh.at[dstv.at[gl, b]], ssem).wait()

    tbase = base_t + (PNG * PGRP) * CH
    pltpu.sync_copy(src_hbm.at[pl.ds(tbase, CH)], srcv.at[0, 0])
    pltpu.sync_copy(dstp_hbm.at[pl.ds(tbase, CH)], dstv.at[0, 0])
    pltpu.async_copy(tab_hbm.at[srcv.at[0, 0]], rows_v.at[0, 0], gsem).wait()
    pltpu.async_copy(rows_v.at[0, 0], acc_sh.at[dstv.at[0, 0]], ssem, add=True)
    pltpu.make_async_copy(rows_v.at[0, 0], acc_sh.at[dstv.at[0, 0]], ssem).wait()

    plsc.subcore_barrier()
    pltpu.sync_copy(acc_sh.at[pl.ds(s * RPT, RPT)], acc_hbm.at[c, pl.ds(s * RPT, RPT)])


EF = 2500


def _tc_fix_body(src_ref, dst_ref, srcp_ref, dstp_ref):
    sv = src_ref[...]
    dv = dst_ref[...]
    ne = sv != dv
    srcp_ref[...] = jnp.where(ne, sv, DUMMY)
    dstp_ref[...] = jnp.where(ne, dv, DUMMY)


def _tc_fix(src2d, dst2d):
    return pl.pallas_call(
        _tc_fix_body,
        out_shape=[
            jax.ShapeDtypeStruct((EF, 128), jnp.int32),
            jax.ShapeDtypeStruct((EF, 128), jnp.int32),
        ],
    )(src2d, dst2d)


def _tc_dinv_body(deg_ref, dinv_ref):
    deg = deg_ref[0:1, :] + deg_ref[1:2, :]
    r = lax.rsqrt(deg)
    dinv_ref[...] = jnp.where(deg > 0.0, r, 0.0)


def _tc_dinv(deg2):
    return pl.pallas_call(
        _tc_dinv_body,
        out_shape=jax.ShapeDtypeStruct((1, NROW), jnp.float32),
    )(deg2)


_BM = 1000


def _tc_xscale_body(x_ref, dinv_ref, xt_ref):
    xt_ref[...] = x_ref[...] * dinv_ref[...]


def _tc_xscale(x, dinv_col):
    return pl.pallas_call(
        _tc_xscale_body,
        grid=(N // _BM,),
        in_specs=[
            pl.BlockSpec((_BM, D), lambda i: (i, 0)),
            pl.BlockSpec((_BM, 1), lambda i: (i, 0)),
        ],
        out_specs=pl.BlockSpec((_BM, D), lambda i: (i, 0)),
        out_shape=jax.ShapeDtypeStruct((N, D), jnp.float32),
    )(x, dinv_col)


def _tc_mid_body(x_ref, u1_ref, dinv_ref, w0_ref, w1_ref, w2_ref, b_ref,
                 p1t_ref, out0_ref):
    dv = dinv_ref[...]
    u1 = u1_ref[0] + u1_ref[1]
    p1 = -dv * u1
    p1t_ref[...] = dv * p1
    w02 = w0_ref[...] - w2_ref[...]
    out0_ref[...] = (
        jnp.dot(x_ref[...], w02, preferred_element_type=jnp.float32)
        + jnp.dot(p1, w1_ref[...], preferred_element_type=jnp.float32)
        + b_ref[...]
    )


def _tc_mid(x, u1parts, dinv_col, W0, W1, W2, bias2d):
    return pl.pallas_call(
        _tc_mid_body,
        grid=(N // _BM,),
        in_specs=[
            pl.BlockSpec((_BM, D), lambda i: (i, 0)),
            pl.BlockSpec((NC, _BM, D), lambda i: (0, i, 0)),
            pl.BlockSpec((_BM, 1), lambda i: (i, 0)),
            pl.BlockSpec((D, D), lambda i: (0, 0)),
            pl.BlockSpec((D, D), lambda i: (0, 0)),
            pl.BlockSpec((D, D), lambda i: (0, 0)),
            pl.BlockSpec((1, D), lambda i: (0, 0)),
        ],
        out_specs=[
            pl.BlockSpec((_BM, D), lambda i: (i, 0)),
            pl.BlockSpec((_BM, D), lambda i: (i, 0)),
        ],
        out_shape=[
            jax.ShapeDtypeStruct((N, D), jnp.float32),
            jax.ShapeDtypeStruct((N, D), jnp.float32),
        ],
    )(x, u1parts, dinv_col, W0, W1, W2, bias2d)


def _tc_final_body(out0_ref, u2_ref, dinv_ref, w2_ref, h_ref):
    u2 = u2_ref[0] + u2_ref[1]
    p2 = -dinv_ref[...] * u2
    out = out0_ref[...] + 2.0 * jnp.dot(p2, w2_ref[...],
                                        preferred_element_type=jnp.float32)
    h_ref[...] = jnp.maximum(out, 0.0)


def _tc_final(out0, u2parts, dinv_col, W2):
    return pl.pallas_call(
        _tc_final_body,
        grid=(N // _BM,),
        in_specs=[
            pl.BlockSpec((_BM, D), lambda i: (i, 0)),
            pl.BlockSpec((NC, _BM, D), lambda i: (0, i, 0)),
            pl.BlockSpec((_BM, 1), lambda i: (i, 0)),
            pl.BlockSpec((D, D), lambda i: (0, 0)),
        ],
        out_specs=pl.BlockSpec((_BM, D), lambda i: (i, 0)),
        out_shape=jax.ShapeDtypeStruct((N, D), jnp.float32),
    )(out0, u2parts, dinv_col, W2)


_FB = _BM * D


def _tc_logits_body(h_ref, lw_ref, lb_ref, logits_ref):
    i = pl.program_id(0)
    contrib = jnp.sum(lw_ref[...] * h_ref[...], axis=1)

    @pl.when(i == 0)
    def _():
        logits_ref[...] = lb_ref[...]

    logits_ref[...] += contrib[None, :]


def _tc_logits(hflat, lin_w, lin_b2d):
    return pl.pallas_call(
        _tc_logits_body,
        grid=(N * D // _FB,),
        in_specs=[
            pl.BlockSpec((1, _FB), lambda i: (0, i)),
            pl.BlockSpec((NCLS, _FB), lambda i: (0, i)),
            pl.BlockSpec((1, NCLS), lambda i: (0, 0)),
        ],
        out_specs=pl.BlockSpec((1, NCLS), lambda i: (0, 0)),
        out_shape=jax.ShapeDtypeStruct((1, NCLS), jnp.float32),
    )(hflat, lin_w, lin_b2d)


def kernel(x, edge_index, batch, W0, W1, W2, conv_bias, lin_w, lin_b):
    src = edge_index[0]
    dst = edge_index[1]

    srcp2d, dstp2d = _tc_fix(src.reshape(EF, 128), dst.reshape(EF, 128))
    srcp = srcp2d.reshape(E)
    dstp = dstp2d.reshape(E)
    deg2 = _sc_deg(srcp)
    dinv_row = _tc_dinv(deg2)
    dinv_col = dinv_row.reshape(NROW, 1)
    xt = _tc_xscale(x, dinv_col)
    u1parts = _sc_prop(src, dstp, xt)
    p1t, out0 = _tc_mid(x, u1parts, dinv_col, W0, W1, W2,
                        conv_bias.reshape(1, D))
    u2parts = _sc_prop(src, dstp, p1t)
    h = _tc_final(out0, u2parts, dinv_col, W2)
    logits = _tc_logits(h.reshape(1, N * D), lin_w, lin_b.reshape(1, NCLS))
    return logits


# speedup vs baseline: 17.1688x; 1.0039x over previous
import functools

import jax
import jax.numpy as jnp
from jax import lax
from jax.experimental import pallas as pl
from jax.experimental.pallas import tpu as pltpu
from jax.experimental.pallas import tpu_sc as plsc

N = 10000
E = 320000
D = 128
NCLS = 10

NC = 2
NS = 16
NW = NC * NS
CH = 80
GRP = 5
EPT = E // NW
NG = EPT // (CH * GRP)
PGRP = 2
PNG = (EPT // CH) // PGRP
NROW = 10240
RPT = NROW // NS
ZB = 640
DUMMY = 10016

_mesh = plsc.VectorSubcoreMesh(core_axis_name="c", subcore_axis_name="s")
_sc_params = pltpu.CompilerParams(needs_layout_passes=False, disable_bounds_checks=True)


@functools.partial(
    pl.kernel,
    out_type=jax.ShapeDtypeStruct((NC, NROW), jnp.float32),
    mesh=_mesh,
    compiler_params=_sc_params,
    scratch_types=[
        pltpu.VMEM((ZB,), jnp.float32),
        pltpu.VMEM((CH,), jnp.float32),
        pltpu.VMEM((2, GRP, CH), jnp.int32),
        pltpu.VMEM_SHARED((NROW,), jnp.float32),
        pltpu.SemaphoreType.DMA,
        pltpu.SemaphoreType.DMA,
    ],
)
def _sc_deg(srcp_hbm, deg_hbm,
            zvec_v, ones_v, srcv, deg_sh, isem, ssem):
    c = lax.axis_index("c")
    s = lax.axis_index("s")
    base_t = c * (E // NC) + s * EPT

    zeros16 = jnp.zeros((16,), jnp.float32)
    ones16 = jnp.ones((16,), jnp.float32)

    def zfill(i, _):
        zvec_v[pl.ds(i * 16, 16)] = zeros16
        return 0

    lax.fori_loop(0, ZB // 16, zfill, 0)
    for j in range(CH // 16):
        ones_v[pl.ds(j * 16, 16)] = ones16

    pltpu.sync_copy(zvec_v, deg_sh.at[pl.ds(s * ZB, ZB)])

    def fire_idx(o, slot):
        for b in range(GRP):
            base = base_t + (o * GRP + b) * CH
            pltpu.async_copy(srcp_hbm.at[pl.ds(base, CH)], srcv.at[slot, b], isem)

    def drain_idx(o, slot):
        for b in range(GRP):
            base = base_t + (o * GRP + b) * CH
            pltpu.make_async_copy(srcp_hbm.at[pl.ds(base, CH)], srcv.at[slot, b], isem).wait()

    fire_idx(0, 0)
    plsc.subcore_barrier()

    def body(o, _):
        g = lax.rem(o, 2)
        g1 = lax.rem(o + 1, 2)
        drain_idx(o, g)

        @pl.when(o > 0)
        def _():
            for b in range(GRP):
                pltpu.make_async_copy(ones_v, deg_sh.at[srcv.at[g1, b]], ssem).wait()

        @pl.when(o < NG - 1)
        def _():
            fire_idx(o + 1, g1)

        for b in range(GRP):
            pltpu.async_copy(ones_v, deg_sh.at[srcv.at[g, b]], ssem, add=True)
        return 0

    lax.fori_loop(0, NG, body, 0)
    for b in range(GRP):
        pltpu.make_async_copy(ones_v, deg_sh.at[srcv.at[0, b]], ssem).wait()
    plsc.subcore_barrier()
    pltpu.sync_copy(deg_sh.at[pl.ds(s * ZB, ZB)], deg_hbm.at[c, pl.ds(s * ZB, ZB)])


@functools.partial(
    pl.kernel,
    out_type=jax.ShapeDtypeStruct((NC, NROW, D), jnp.float32),
    mesh=_mesh,
    compiler_params=_sc_params,
    scratch_types=[
        pltpu.VMEM((32, D), jnp.float32),
        pltpu.VMEM((2, PGRP, CH), jnp.int32),
        pltpu.VMEM((2, PGRP, CH), jnp.int32),
        pltpu.VMEM((2, PGRP, CH, D), jnp.float32),
        pltpu.VMEM_SHARED((NROW, D), jnp.float32),
        pltpu.SemaphoreType.DMA,
        pltpu.SemaphoreType.DMA,
        pltpu.SemaphoreType.DMA,
    ],
)
def _sc_prop(src_hbm, dstp_hbm, tab_hbm, acc_hbm,
             zrows_v, srcv, dstv, rows_v, acc_sh, isem, gsem, ssem):
    c = lax.axis_index("c")
    s = lax.axis_index("s")
    wid = c * NS + s
    base_t = wid * EPT

    zeros16 = jnp.zeros((16,), jnp.float32)

    def zfill(i, _):
        for cc in range(8):
            zrows_v[i, pl.ds(cc * 16, 16)] = zeros16
        return 0

    lax.fori_loop(0, 32, zfill, 0)
    for k in range(20):
        pltpu.async_copy(zrows_v, acc_sh.at[pl.ds(s * RPT + k * 32, 32)], ssem)
    for k in range(20):
        pltpu.make_async_copy(zrows_v, acc_sh.at[pl.ds(s * RPT + k * 32, 32)], ssem).wait()

    def fire_idx(o, slot):
        for b in range(PGRP):
            base = base_t + (o * PGRP + b) * CH
            pltpu.async_copy(src_hbm.at[pl.ds(base, CH)], srcv.at[slot, b], isem)
            pltpu.async_copy(dstp_hbm.at[pl.ds(base, CH)], dstv.at[slot, b], isem)

    def drain_idx(o, slot):
        for b in range(PGRP):
            base = base_t + (o * PGRP + b) * CH
            pltpu.make_async_copy(src_hbm.at[pl.ds(base, CH)], srcv.at[slot, b], isem).wait()
            pltpu.make_async_copy(dstp_hbm.at[pl.ds(base, CH)], dstv.at[slot, b], isem).wait()

    fire_idx(0, 0)
    plsc.subcore_barrier()

    def body(o, _):
        g = lax.rem(o, 2)
        g1 = lax.rem(o + 1, 2)
        drain_idx(o, g)

        @pl.when(o > 0)
        def _():
            for b in range(PGRP):
                pltpu.make_async_copy(rows_v.at[g1, b], acc_sh.at[dstv.at[g1, b]], ssem).wait()

        for b in range(PGRP):
            pltpu.async_copy(tab_hbm.at[srcv.at[g, b]], rows_v.at[g, b], gsem)

        @pl.when(o < PNG - 1)
        def _():
            fire_idx(o + 1, g1)

        for b in range(PGRP):
            pltpu.make_async_copy(tab_hbm.at[srcv.at[g, b]], rows_v.at[g, b], gsem).wait()
        for b in range(PGRP):
            pltpu.async_copy(rows_v.at[g, b], acc_sh.at[dstv.at[g, b]], ssem, add=True)
        return 0

    lax.fori_loop(0, PNG, body, 0)
    gl = (PNG - 1) % 2
    for b in range(PGRP):
        pltpu.make_async_copy(rows_v.at[gl, b], acc_sh.at[dstv.at[gl, b]], ssem).wait()

    tbase = base_t + (PNG * PGRP) * CH
    pltpu.sync_copy(src_hbm.at[pl.ds(tbase, CH)], srcv.at[0, 0])
    pltpu.sync_copy(dstp_hbm.at[pl.ds(tbase, CH)], dstv.at[0, 0])
    pltpu.async_copy(tab_hbm.at[srcv.at[0, 0]], rows_v.at[0, 0], gsem).wait()
    pltpu.async_copy(rows_v.at[0, 0], acc_sh.at[dstv.at[0, 0]], ssem, add=True)
    pltpu.make_async_copy(rows_v.at[0, 0], acc_sh.at[dstv.at[0, 0]], ssem).wait()

    plsc.subcore_barrier()
    pltpu.sync_copy(acc_sh.at[pl.ds(s * RPT, RPT)], acc_hbm.at[c, pl.ds(s * RPT, RPT)])


EF = 2500


def _tc_fix_body(src_ref, dst_ref, srcp_ref, dstp_ref):
    sv = src_ref[...]
    dv = dst_ref[...]
    ne = sv != dv
    srcp_ref[...] = jnp.where(ne, sv, DUMMY)
    dstp_ref[...] = jnp.where(ne, dv, DUMMY)


def _tc_fix(src2d, dst2d):
    return pl.pallas_call(
        _tc_fix_body,
        out_shape=[
            jax.ShapeDtypeStruct((EF, 128), jnp.int32),
            jax.ShapeDtypeStruct((EF, 128), jnp.int32),
        ],
    )(src2d, dst2d)


def _tc_dinv_body(deg_ref, dinv_ref):
    deg = deg_ref[0:1, :] + deg_ref[1:2, :]
    r = lax.rsqrt(deg)
    dinv_ref[...] = jnp.where(deg > 0.0, r, 0.0)


def _tc_dinv(deg2):
    return pl.pallas_call(
        _tc_dinv_body,
        out_shape=jax.ShapeDtypeStruct((1, NROW), jnp.float32),
    )(deg2)


_BM = 1000


def _tc_xscale_body(x_ref, dinv_ref, xt_ref):
    xt_ref[...] = x_ref[...] * dinv_ref[...]


def _tc_xscale(x, dinv_col):
    return pl.pallas_call(
        _tc_xscale_body,
        grid=(N // _BM,),
        in_specs=[
            pl.BlockSpec((_BM, D), lambda i: (i, 0)),
            pl.BlockSpec((_BM, 1), lambda i: (i, 0)),
        ],
        out_specs=pl.BlockSpec((_BM, D), lambda i: (i, 0)),
        out_shape=jax.ShapeDtypeStruct((N, D), jnp.float32),
    )(x, dinv_col)


def _tc_mid_body(x_ref, u1_ref, dinv_ref, w0_ref, w1_ref, w2_ref, b_ref,
                 p1t_ref, out0_ref):
    dv = dinv_ref[...]
    u1 = u1_ref[0] + u1_ref[1]
    p1 = -dv * u1
    p1t_ref[...] = dv * p1
    w02 = w0_ref[...] - w2_ref[...]
    out0_ref[...] = (
        jnp.dot(x_ref[...], w02, preferred_element_type=jnp.float32)
        + jnp.dot(p1, w1_ref[...], preferred_element_type=jnp.float32)
        + b_ref[...]
    )


def _tc_mid(x, u1parts, dinv_col, W0, W1, W2, bias2d):
    return pl.pallas_call(
        _tc_mid_body,
        grid=(N // _BM,),
        in_specs=[
            pl.BlockSpec((_BM, D), lambda i: (i, 0)),
            pl.BlockSpec((NC, _BM, D), lambda i: (0, i, 0)),
            pl.BlockSpec((_BM, 1), lambda i: (i, 0)),
            pl.BlockSpec((D, D), lambda i: (0, 0)),
            pl.BlockSpec((D, D), lambda i: (0, 0)),
            pl.BlockSpec((D, D), lambda i: (0, 0)),
            pl.BlockSpec((1, D), lambda i: (0, 0)),
        ],
        out_specs=[
            pl.BlockSpec((_BM, D), lambda i: (i, 0)),
            pl.BlockSpec((_BM, D), lambda i: (i, 0)),
        ],
        out_shape=[
            jax.ShapeDtypeStruct((N, D), jnp.float32),
            jax.ShapeDtypeStruct((N, D), jnp.float32),
        ],
    )(x, u1parts, dinv_col, W0, W1, W2, bias2d)


def _tc_final_body(out0_ref, u2_ref, dinv_ref, w2_ref, h_ref):
    u2 = u2_ref[0] + u2_ref[1]
    p2 = -dinv_ref[...] * u2
    out = out0_ref[...] + 2.0 * jnp.dot(p2, w2_ref[...],
                                        preferred_element_type=jnp.float32)
    h_ref[...] = jnp.maximum(out, 0.0)


def _tc_final(out0, u2parts, dinv_col, W2):
    return pl.pallas_call(
        _tc_final_body,
        grid=(N // _BM,),
        in_specs=[
            pl.BlockSpec((_BM, D), lambda i: (i, 0)),
            pl.BlockSpec((NC, _BM, D), lambda i: (0, i, 0)),
            pl.BlockSpec((_BM, 1), lambda i: (i, 0)),
            pl.BlockSpec((D, D), lambda i: (0, 0)),
        ],
        out_specs=pl.BlockSpec((_BM, D), lambda i: (i, 0)),
        out_shape=jax.ShapeDtypeStruct((N, D), jnp.float32),
    )(out0, u2parts, dinv_col, W2)


_FB = _BM * D


def _tc_logits_body(h_ref, lw_ref, lb_ref, logits_ref):
    i = pl.program_id(0)
    contrib = jnp.sum(lw_ref[...] * h_ref[...], axis=1)

    @pl.when(i == 0)
    def _():
        logits_ref[...] = lb_ref[...]

    logits_ref[...] += contrib[None, :]


def _tc_logits(hflat, lin_w, lin_b2d):
    return pl.pallas_call(
        _tc_logits_body,
        grid=(N * D // _FB,),
        in_specs=[
            pl.BlockSpec((1, _FB), lambda i: (0, i)),
            pl.BlockSpec((NCLS, _FB), lambda i: (0, i)),
            pl.BlockSpec((1, NCLS), lambda i: (0, 0)),
        ],
        out_specs=pl.BlockSpec((1, NCLS), lambda i: (0, 0)),
        out_shape=jax.ShapeDtypeStruct((1, NCLS), jnp.float32),
    )(hflat, lin_w, lin_b2d)


def kernel(x, edge_index, batch, W0, W1, W2, conv_bias, lin_w, lin_b):
    src = edge_index[0]
    dst = edge_index[1]

    srcp2d, dstp2d = _tc_fix(src.reshape(EF, 128), dst.reshape(EF, 128))
    srcp = srcp2d.reshape(E)
    dstp = dstp2d.reshape(E)
    deg2 = _sc_deg(srcp)
    dinv_row = _tc_dinv(deg2)
    dinv_col = dinv_row.reshape(NROW, 1)
    xt = _tc_xscale(x, dinv_col)
    u1parts = _sc_prop(src, dstp, xt)
    p1t, out0 = _tc_mid(x, u1parts, dinv_col, W0, W1, W2,
                        conv_bias.reshape(1, D))
    u2parts = _sc_prop(src, dstp, p1t)
    h = _tc_final(out0, u2parts, dinv_col, W2)
    logits = _tc_logits(h.reshape(1, N * D), lin_w, lin_b.reshape(1, NCLS))
    return logits
